# baseline (speedup 1.0000x reference)
import functools

import jax
import jax.numpy as jnp
from jax import lax
from jax.experimental import pallas as pl
from jax.experimental.pallas import tpu as pltpu
from jax.experimental.pallas import tpu_sc as plsc

N = 10000
NP = 10240
E = 160000
NC, NS, L = 2, 16, 16
NW = NC * NS
CHUNK = 128
NCH = 40
EPT = NCH * CHUNK
EP = NW * EPT
RPT = NP // NS
HALF = EPT // 2

_mesh = plsc.VectorSubcoreMesh(
    core_axis_name="c", subcore_axis_name="s", num_cores=NC, num_subcores=NS
)
_sc_params = pltpu.CompilerParams(use_tc_tiling_on_sc=False,
                                  needs_layout_passes=False)


def _rsqrt16(x):
    i = plsc.bitcast(x, jnp.int32)
    i = jnp.full((L,), 0x5F3759DF, jnp.int32) - lax.shift_right_logical(i, 1)
    y = plsc.bitcast(i, jnp.float32)
    for _ in range(2):
        y = y * (1.5 - 0.5 * x * y * y)
    r = 1.0 - x * y * y
    return y * (1.0 + r * (0.5 + 0.375 * r))


def _zero_into(zbuf, agg_sh, s):
    @plsc.parallel_loop(0, RPT, unroll=8)
    def zbody(i):
        zbuf[i] = jnp.zeros((L,), jnp.float32)

    pltpu.sync_copy(zbuf.at[pl.ds(0, RPT)], agg_sh.at[pl.ds(s * RPT, RPT)])


def _readout(obuf, agg_sh, out_hbm, c, s):
    pltpu.sync_copy(agg_sh.at[pl.ds(s * RPT, RPT)], obuf)
    pltpu.sync_copy(obuf, out_hbm.at[c, pl.ds(s * RPT, RPT)])


def _edge_sweep(row_v, col_v, gbufa, gbufb, h_sh, agg_sh, gsema, gsemb, ssem):
    pltpu.async_copy(h_sh.at[row_v.at[pl.ds(0, HALF)]], gbufa, gsema)
    pltpu.async_copy(h_sh.at[row_v.at[pl.ds(HALF, HALF)]], gbufb, gsemb)
    pltpu.make_async_copy(h_sh.at[row_v.at[pl.ds(0, HALF)]], gbufa,
                          gsema).wait()

    def firea(j, carry):
        pltpu.async_copy(gbufa.at[pl.ds(j * CHUNK, CHUNK)],
                         agg_sh.at[col_v.at[j]], ssem, add=True)
        return carry

    lax.fori_loop(0, NCH // 2, firea, 0)
    pltpu.make_async_copy(h_sh.at[row_v.at[pl.ds(HALF, HALF)]], gbufb,
                          gsemb).wait()

    def fireb(j, carry):
        pltpu.async_copy(gbufb.at[pl.ds((j - NCH // 2) * CHUNK, CHUNK)],
                         agg_sh.at[col_v.at[j]], ssem, add=True)
        return carry

    lax.fori_loop(NCH // 2, NCH, fireb, 0)

    def drain(j, carry):
        pltpu.make_async_copy(gbufa.at[pl.ds(0, CHUNK)],
                              agg_sh.at[col_v.at[j]], ssem).wait()
        return carry

    lax.fori_loop(0, NCH, drain, 0)


def _sc_deg_body(ones_hbm, col_hbm, out_hbm, col_v, ones_v, obuf, deg_sh,
                 ssem):
    c = lax.axis_index("c")
    s = lax.axis_index("s")
    wid = c * NS + s
    pltpu.sync_copy(col_hbm.at[wid], col_v)
    pltpu.sync_copy(ones_hbm, ones_v)

    @plsc.parallel_loop(0, RPT // L, unroll=8)
    def zbody(i):
        obuf[pl.ds(i * L, L)] = jnp.zeros((L,), jnp.float32)

    pltpu.sync_copy(obuf, deg_sh.at[pl.ds(s * RPT, RPT)])
    plsc.subcore_barrier()

    def fire(j, carry):
        pltpu.async_copy(ones_v, deg_sh.at[col_v.at[j]], ssem, add=True)
        return carry

    lax.fori_loop(0, NCH, fire, 0)

    def drain(j, carry):
        pltpu.make_async_copy(ones_v, deg_sh.at[col_v.at[j]], ssem).wait()
        return carry

    lax.fori_loop(0, NCH, drain, 0)
    plsc.subcore_barrier()
    pltpu.sync_copy(deg_sh.at[pl.ds(s * RPT, RPT)], obuf)
    pltpu.sync_copy(obuf, out_hbm.at[c, pl.ds(s * RPT, RPT)])


_sc_deg = functools.partial(
    pl.kernel,
    out_type=jax.ShapeDtypeStruct((NC, NP), jnp.float32),
    mesh=_mesh,
    compiler_params=_sc_params,
    scratch_types=[
        pltpu.VMEM((NCH, CHUNK), jnp.int32),
        pltpu.VMEM((CHUNK,), jnp.float32),
        pltpu.VMEM((RPT,), jnp.float32),
        pltpu.VMEM_SHARED((NP,), jnp.float32),
        pltpu.SemaphoreType.DMA,
    ],
)(_sc_deg_body)


def _sc_agg1_body(h1_hbm, dp_hbm, row_hbm, col_hbm, p_hbm, h1s_hbm, dv_hbm,
                  row_v, col_v, gbufa, gbufb, obuf, dbuf, h_sh, agg_sh,
                  gsema, gsemb, ssem):
    c = lax.axis_index("c")
    s = lax.axis_index("s")
    wid = c * NS + s
    pltpu.sync_copy(row_hbm.at[wid], row_v)
    pltpu.sync_copy(col_hbm.at[wid], col_v)
    sl = pl.ds(s * RPT, RPT)
    pltpu.async_copy(h1_hbm.at[sl], gbufa.at[pl.ds(0, RPT)], gsema)
    pltpu.async_copy(dp_hbm.at[0, sl], dbuf.at[pl.ds(0, RPT)], gsema)
    pltpu.async_copy(dp_hbm.at[1, sl], dbuf.at[pl.ds(RPT, RPT)], gsema)
    _zero_into(gbufb, agg_sh, s)
    pltpu.make_async_copy(h1_hbm.at[sl], gbufa.at[pl.ds(0, RPT)],
                          gsema).wait()
    pltpu.make_async_copy(dp_hbm.at[0, sl], dbuf.at[pl.ds(0, RPT)],
                          gsema).wait()
    pltpu.make_async_copy(dp_hbm.at[1, sl], dbuf.at[pl.ds(RPT, RPT)],
                          gsema).wait()

    @plsc.parallel_loop(0, RPT // L, unroll=2)
    def stage(g):
        dvec = dbuf[pl.ds(g * L, L)] + dbuf[pl.ds(RPT + g * L, L)] + 1.0
        dinv16 = _rsqrt16(dvec)
        for k in range(L):
            i = g * L + k
            d = jnp.broadcast_to(dinv16[k], (L,))
            obuf[i] = d * gbufa[i]
            gbufa[RPT + i] = d
    pltpu.sync_copy(obuf, h_sh.at[sl])

    @pl.when(c == 0)
    def _():
        pltpu.sync_copy(obuf, h1s_hbm.at[sl])
        pltpu.sync_copy(gbufa.at[pl.ds(RPT, RPT)], dv_hbm.at[sl])

    plsc.subcore_barrier()
    _edge_sweep(row_v, col_v, gbufa, gbufb, h_sh, agg_sh, gsema, gsemb, ssem)
    plsc.subcore_barrier()
    _readout(obuf, agg_sh, p_hbm, c, s)


def _sc_agg2_body(p_hbm, h1s_hbm, dv_hbm, b1_hbm, row_hbm, col_hbm,
                  q_hbm, u_hbm, row_v, col_v, gbufa, gbufb, obuf, bv,
                  h_sh, agg_sh, gsema, gsemb, ssem):
    c = lax.axis_index("c")
    s = lax.axis_index("s")
    wid = c * NS + s
    pltpu.sync_copy(row_hbm.at[wid], row_v)
    pltpu.sync_copy(col_hbm.at[wid], col_v)
    pltpu.sync_copy(b1_hbm, bv)
    sl = pl.ds(s * RPT, RPT)
    pltpu.async_copy(p_hbm.at[0, sl], gbufa.at[pl.ds(0, RPT)], gsema)
    pltpu.async_copy(p_hbm.at[1, sl], gbufa.at[pl.ds(RPT, RPT)], gsema)
    pltpu.async_copy(h1s_hbm.at[sl], gbufa.at[pl.ds(2 * RPT, RPT)], gsema)
    pltpu.async_copy(dv_hbm.at[sl], gbufa.at[pl.ds(3 * RPT, RPT)], gsema)
    _zero_into(gbufb, agg_sh, s)
    for _ in range(4):
        pltpu.make_async_copy(p_hbm.at[0, sl], gbufa.at[pl.ds(0, RPT)],
                              gsema).wait()
    bval = bv[...]

    @plsc.parallel_loop(0, RPT, unroll=8)
    def stage(i):
        d = gbufa[3 * RPT + i]
        t = (gbufa[i] + gbufa[RPT + i] + gbufa[2 * RPT + i]) * d + bval
        obuf[i] = jnp.maximum(t, 0.0) * d
    pltpu.sync_copy(obuf, h_sh.at[sl])

    @pl.when(c == 0)
    def _():
        pltpu.sync_copy(obuf, u_hbm.at[sl])

    plsc.subcore_barrier()
    _edge_sweep(row_v, col_v, gbufa, gbufb, h_sh, agg_sh, gsema, gsemb, ssem)
    plsc.subcore_barrier()
    _readout(obuf, agg_sh, q_hbm, c, s)


_agg_scratch = [
    pltpu.VMEM((EPT,), jnp.int32),
    pltpu.VMEM((NCH, CHUNK), jnp.int32),
    pltpu.VMEM((HALF, L), jnp.float32),
    pltpu.VMEM((HALF, L), jnp.float32),
    pltpu.VMEM((RPT, L), jnp.float32),
    pltpu.VMEM_SHARED((NP, L), jnp.float32),
    pltpu.VMEM_SHARED((NP, L), jnp.float32),
    pltpu.SemaphoreType.DMA,
    pltpu.SemaphoreType.DMA,
    pltpu.SemaphoreType.DMA,
]

_sc_agg1 = functools.partial(
    pl.kernel,
    out_type=(
        jax.ShapeDtypeStruct((NC, NP, L), jnp.float32),
        jax.ShapeDtypeStruct((NP, L), jnp.float32),
        jax.ShapeDtypeStruct((NP, L), jnp.float32),
    ),
    mesh=_mesh,
    compiler_params=_sc_params,
    scratch_types=_agg_scratch[:5] + [
        pltpu.VMEM((2 * RPT,), jnp.float32),
    ] + _agg_scratch[5:],
)(_sc_agg1_body)

_sc_agg2 = functools.partial(
    pl.kernel,
    out_type=(
        jax.ShapeDtypeStruct((NC, NP, L), jnp.float32),
        jax.ShapeDtypeStruct((NP, L), jnp.float32),
    ),
    mesh=_mesh,
    compiler_params=_sc_params,
    scratch_types=_agg_scratch[:4] + [
        pltpu.VMEM((RPT, L), jnp.float32),
        pltpu.VMEM((L,), jnp.float32),
    ] + _agg_scratch[5:],
)(_sc_agg2_body)


BN = 2000
GRID = N // BN


def _mm_body(x_ref, w_ref, o_ref):
    o_ref[...] = jnp.dot(x_ref[...], w_ref[...],
                         preferred_element_type=jnp.float32)


_tc_mm = pl.pallas_call(
    _mm_body,
    grid=(GRID,),
    in_specs=[
        pl.BlockSpec((BN, 256), lambda i: (i, 0)),
        pl.BlockSpec((256, L), lambda i: (0, 0)),
    ],
    out_specs=pl.BlockSpec((BN, L), lambda i: (i, 0)),
    out_shape=jax.ShapeDtypeStruct((N, L), jnp.float32),
)


def _tc3_body(q_hbm, u_hbm, d_hbm, w_ref, b_ref, o_ref, qv, uv, dv, sem):
    i = pl.program_id(0)
    rows = pl.ds(i * BN, BN)
    pltpu.make_async_copy(q_hbm.at[0, rows], qv.at[0], sem).start()
    pltpu.make_async_copy(q_hbm.at[1, rows], qv.at[1], sem).start()
    pltpu.make_async_copy(u_hbm.at[rows], uv, sem).start()
    pltpu.make_async_copy(d_hbm.at[rows], dv, sem).start()
    pltpu.make_async_copy(q_hbm.at[0, rows], qv.at[0], sem).wait()
    pltpu.make_async_copy(q_hbm.at[1, rows], qv.at[1], sem).wait()
    pltpu.make_async_copy(u_hbm.at[rows], uv, sem).wait()
    pltpu.make_async_copy(d_hbm.at[rows], dv, sem).wait()
    agg = qv[0] + qv[1] + uv[...]
    t = jnp.dot(agg * dv[...], w_ref[...],
                preferred_element_type=jnp.float32)
    t = t + b_ref[...]
    m = jnp.max(t, axis=1, keepdims=True)
    e = t - m
    o_ref[...] = e - jnp.log(jnp.sum(jnp.exp(e), axis=1, keepdims=True))


_tc3 = pl.pallas_call(
    _tc3_body,
    grid=(GRID,),
    in_specs=[
        pl.BlockSpec(memory_space=pl.ANY),
        pl.BlockSpec(memory_space=pl.ANY),
        pl.BlockSpec(memory_space=pl.ANY),
        pl.BlockSpec((L, 64), lambda i: (0, 0)),
        pl.BlockSpec((1, 64), lambda i: (0, 0)),
    ],
    out_specs=pl.BlockSpec((BN, 64), lambda i: (i, 0)),
    out_shape=jax.ShapeDtypeStruct((N, 64), jnp.float32),
    scratch_shapes=[
        pltpu.VMEM((NC, BN, L), jnp.float32),
        pltpu.VMEM((BN, L), jnp.float32),
        pltpu.VMEM((BN, L), jnp.float32),
        pltpu.SemaphoreType.DMA,
    ],
)


def kernel(x, edge_index, W1, b1, W2, b2):
    spread = N + (jnp.arange(EP - E, dtype=jnp.int32) % (NP - N))
    row3 = jnp.concatenate([edge_index[0], spread]).reshape(NW, EPT)
    col3 = jnp.concatenate([edge_index[1], spread]).reshape(NW, NCH, CHUNK)
    ones_src = jnp.ones((CHUNK,), jnp.float32)

    dpart = _sc_deg(ones_src, col3)
    h1 = _tc_mm(x, W1)
    h1p = jnp.pad(h1, ((0, NP - N), (0, 0)))
    p, h1s, dv = _sc_agg1(h1p, dpart, row3, col3)
    q, u = _sc_agg2(p, h1s, dv, b1, row3, col3)
    return _tc3(q, u, dv, W2, b2.reshape(1, 64))


# speedup vs baseline: 1.0737x; 1.0737x over previous
import functools

import jax
import jax.numpy as jnp
from jax import lax
from jax.experimental import pallas as pl
from jax.experimental.pallas import tpu as pltpu
from jax.experimental.pallas import tpu_sc as plsc

N = 10000
NP = 10240
E = 160000
NC, NS, L = 2, 16, 16
NW = NC * NS
CHUNK = 128
NCH = 40
EPT = NCH * CHUNK
EP = NW * EPT
RPT = NP // NS
HALF = EPT // 2

_mesh = plsc.VectorSubcoreMesh(
    core_axis_name="c", subcore_axis_name="s", num_cores=NC, num_subcores=NS
)
_sc_params = pltpu.CompilerParams(use_tc_tiling_on_sc=False,
                                  needs_layout_passes=False)


def _rsqrt16(x):
    i = plsc.bitcast(x, jnp.int32)
    i = jnp.full((L,), 0x5F3759DF, jnp.int32) - lax.shift_right_logical(i, 1)
    y = plsc.bitcast(i, jnp.float32)
    for _ in range(2):
        y = y * (1.5 - 0.5 * x * y * y)
    r = 1.0 - x * y * y
    return y * (1.0 + r * (0.5 + 0.375 * r))


def _zero_into(zbuf, agg_sh, s):
    @plsc.parallel_loop(0, RPT, unroll=8)
    def zbody(i):
        zbuf[i] = jnp.zeros((L,), jnp.float32)

    pltpu.sync_copy(zbuf.at[pl.ds(0, RPT)], agg_sh.at[pl.ds(s * RPT, RPT)])


def _readout(obuf, agg_sh, out_hbm, c, s):
    pltpu.sync_copy(agg_sh.at[pl.ds(s * RPT, RPT)], obuf)
    pltpu.sync_copy(obuf, out_hbm.at[c, pl.ds(s * RPT, RPT)])


def _edge_sweep(row_v, col_v, gbufa, gbufb, h_sh, agg_sh, gsema, gsemb, ssem):
    pltpu.async_copy(h_sh.at[row_v.at[pl.ds(0, HALF)]], gbufa, gsema)
    pltpu.async_copy(h_sh.at[row_v.at[pl.ds(HALF, HALF)]], gbufb, gsemb)
    pltpu.make_async_copy(h_sh.at[row_v.at[pl.ds(0, HALF)]], gbufa,
                          gsema).wait()

    def firea(j, carry):
        pltpu.async_copy(gbufa.at[pl.ds(j * CHUNK, CHUNK)],
                         agg_sh.at[col_v.at[j]], ssem, add=True)
        return carry

    lax.fori_loop(0, NCH // 2, firea, 0)
    pltpu.make_async_copy(h_sh.at[row_v.at[pl.ds(HALF, HALF)]], gbufb,
                          gsemb).wait()

    def fireb(j, carry):
        pltpu.async_copy(gbufb.at[pl.ds((j - NCH // 2) * CHUNK, CHUNK)],
                         agg_sh.at[col_v.at[j]], ssem, add=True)
        return carry

    lax.fori_loop(NCH // 2, NCH, fireb, 0)

    def drain(j, carry):
        pltpu.make_async_copy(gbufa.at[pl.ds(0, CHUNK)],
                              agg_sh.at[col_v.at[j]], ssem).wait()
        return carry

    lax.fori_loop(0, NCH, drain, 0)


def _sc_deg_body(ones_hbm, col_hbm, out_hbm, col_v, ones_v, obuf, deg_sh,
                 ssem):
    c = lax.axis_index("c")
    s = lax.axis_index("s")
    wid = c * NS + s
    pltpu.sync_copy(col_hbm.at[wid], col_v)
    pltpu.sync_copy(ones_hbm, ones_v)

    @plsc.parallel_loop(0, RPT // L, unroll=8)
    def zbody(i):
        obuf[pl.ds(i * L, L)] = jnp.zeros((L,), jnp.float32)

    pltpu.sync_copy(obuf, deg_sh.at[pl.ds(s * RPT, RPT)])
    plsc.subcore_barrier()

    def fire(j, carry):
        pltpu.async_copy(ones_v, deg_sh.at[col_v.at[j]], ssem, add=True)
        return carry

    lax.fori_loop(0, NCH, fire, 0)

    def drain(j, carry):
        pltpu.make_async_copy(ones_v, deg_sh.at[col_v.at[j]], ssem).wait()
        return carry

    lax.fori_loop(0, NCH, drain, 0)
    plsc.subcore_barrier()
    pltpu.sync_copy(deg_sh.at[pl.ds(s * RPT, RPT)], obuf)
    pltpu.sync_copy(obuf, out_hbm.at[c, pl.ds(s * RPT, RPT)])


_sc_deg = functools.partial(
    pl.kernel,
    out_type=jax.ShapeDtypeStruct((NC, NP), jnp.float32),
    mesh=_mesh,
    compiler_params=_sc_params,
    scratch_types=[
        pltpu.VMEM((NCH, CHUNK), jnp.int32),
        pltpu.VMEM((CHUNK,), jnp.float32),
        pltpu.VMEM((RPT,), jnp.float32),
        pltpu.VMEM_SHARED((NP,), jnp.float32),
        pltpu.SemaphoreType.DMA,
    ],
)(_sc_deg_body)


def _sc_agg1_body(h1_hbm, dp_hbm, row_hbm, col_hbm, p_hbm, h1s_hbm, dv_hbm,
                  row_v, col_v, gbufa, gbufb, obuf, dbuf, h_sh, agg_sh,
                  gsema, gsemb, ssem):
    c = lax.axis_index("c")
    s = lax.axis_index("s")
    wid = c * NS + s
    pltpu.sync_copy(row_hbm.at[wid], row_v)
    pltpu.sync_copy(col_hbm.at[wid], col_v)
    sl = pl.ds(s * RPT, RPT)
    pltpu.async_copy(h1_hbm.at[sl], gbufa.at[pl.ds(0, RPT)], gsema)
    pltpu.async_copy(dp_hbm.at[0, sl], dbuf.at[pl.ds(0, RPT)], gsema)
    pltpu.async_copy(dp_hbm.at[1, sl], dbuf.at[pl.ds(RPT, RPT)], gsema)
    _zero_into(gbufb, agg_sh, s)
    pltpu.make_async_copy(h1_hbm.at[sl], gbufa.at[pl.ds(0, RPT)],
                          gsema).wait()
    pltpu.make_async_copy(dp_hbm.at[0, sl], dbuf.at[pl.ds(0, RPT)],
                          gsema).wait()
    pltpu.make_async_copy(dp_hbm.at[1, sl], dbuf.at[pl.ds(RPT, RPT)],
                          gsema).wait()

    @plsc.parallel_loop(0, RPT // L, unroll=2)
    def stage(g):
        dvec = dbuf[pl.ds(g * L, L)] + dbuf[pl.ds(RPT + g * L, L)] + 1.0
        dinv16 = _rsqrt16(dvec)
        for k in range(L):
            i = g * L + k
            d = jnp.broadcast_to(dinv16[k], (L,))
            obuf[i] = d * gbufa[i]
            gbufa[RPT + i] = d
    pltpu.sync_copy(obuf, h_sh.at[sl])

    @pl.when(c == 0)
    def _():
        pltpu.sync_copy(obuf, h1s_hbm.at[sl])
        pltpu.sync_copy(gbufa.at[pl.ds(RPT, RPT)], dv_hbm.at[sl])

    plsc.subcore_barrier()
    _edge_sweep(row_v, col_v, gbufa, gbufb, h_sh, agg_sh, gsema, gsemb, ssem)
    plsc.subcore_barrier()
    _readout(obuf, agg_sh, p_hbm, c, s)


def _sc_agg2_body(p_hbm, h1s_hbm, dv_hbm, b1_hbm, row_hbm, col_hbm,
                  q_hbm, u_hbm, row_v, col_v, gbufa, gbufb, obuf, bv,
                  h_sh, agg_sh, gsema, gsemb, ssem):
    c = lax.axis_index("c")
    s = lax.axis_index("s")
    wid = c * NS + s
    pltpu.sync_copy(row_hbm.at[wid], row_v)
    pltpu.sync_copy(col_hbm.at[wid], col_v)
    pltpu.sync_copy(b1_hbm, bv)
    sl = pl.ds(s * RPT, RPT)
    pltpu.async_copy(p_hbm.at[0, sl], gbufa.at[pl.ds(0, RPT)], gsema)
    pltpu.async_copy(p_hbm.at[1, sl], gbufa.at[pl.ds(RPT, RPT)], gsema)
    pltpu.async_copy(h1s_hbm.at[sl], gbufa.at[pl.ds(2 * RPT, RPT)], gsema)
    pltpu.async_copy(dv_hbm.at[sl], gbufa.at[pl.ds(3 * RPT, RPT)], gsema)
    _zero_into(gbufb, agg_sh, s)
    for _ in range(4):
        pltpu.make_async_copy(p_hbm.at[0, sl], gbufa.at[pl.ds(0, RPT)],
                              gsema).wait()
    bval = bv[...]

    @plsc.parallel_loop(0, RPT, unroll=8)
    def stage(i):
        d = gbufa[3 * RPT + i]
        t = (gbufa[i] + gbufa[RPT + i] + gbufa[2 * RPT + i]) * d + bval
        obuf[i] = jnp.maximum(t, 0.0) * d
    pltpu.sync_copy(obuf, h_sh.at[sl])

    @pl.when(c == 0)
    def _():
        pltpu.sync_copy(obuf, u_hbm.at[sl])

    plsc.subcore_barrier()
    _edge_sweep(row_v, col_v, gbufa, gbufb, h_sh, agg_sh, gsema, gsemb, ssem)
    plsc.subcore_barrier()
    _readout(obuf, agg_sh, q_hbm, c, s)


_agg_scratch = [
    pltpu.VMEM((EPT,), jnp.int32),
    pltpu.VMEM((NCH, CHUNK), jnp.int32),
    pltpu.VMEM((HALF, L), jnp.float32),
    pltpu.VMEM((HALF, L), jnp.float32),
    pltpu.VMEM((RPT, L), jnp.float32),
    pltpu.VMEM_SHARED((NP, L), jnp.float32),
    pltpu.VMEM_SHARED((NP, L), jnp.float32),
    pltpu.SemaphoreType.DMA,
    pltpu.SemaphoreType.DMA,
    pltpu.SemaphoreType.DMA,
]

_sc_agg1 = functools.partial(
    pl.kernel,
    out_type=(
        jax.ShapeDtypeStruct((NC, NP, L), jnp.float32),
        jax.ShapeDtypeStruct((NP, L), jnp.float32),
        jax.ShapeDtypeStruct((NP, L), jnp.float32),
    ),
    mesh=_mesh,
    compiler_params=_sc_params,
    scratch_types=_agg_scratch[:5] + [
        pltpu.VMEM((2 * RPT,), jnp.float32),
    ] + _agg_scratch[5:],
)(_sc_agg1_body)

_sc_agg2 = functools.partial(
    pl.kernel,
    out_type=(
        jax.ShapeDtypeStruct((NC, NP, L), jnp.float32),
        jax.ShapeDtypeStruct((NP, L), jnp.float32),
    ),
    mesh=_mesh,
    compiler_params=_sc_params,
    scratch_types=_agg_scratch[:4] + [
        pltpu.VMEM((RPT, L), jnp.float32),
        pltpu.VMEM((L,), jnp.float32),
    ] + _agg_scratch[5:],
)(_sc_agg2_body)


BN = 2000
GRID = N // BN


def _mm_body(x_ref, w_ref, o_ref):
    o_ref[...] = jnp.dot(x_ref[...], w_ref[...],
                         preferred_element_type=jnp.float32)


_tc_mm = pl.pallas_call(
    _mm_body,
    grid=(GRID,),
    in_specs=[
        pl.BlockSpec((BN, 256), lambda i: (i, 0)),
        pl.BlockSpec((256, L), lambda i: (0, 0)),
    ],
    out_specs=pl.BlockSpec((BN, L), lambda i: (i, 0)),
    out_shape=jax.ShapeDtypeStruct((N, L), jnp.float32),
)


def _tc3_body(q_ref, u_ref, d_ref, w_ref, b_ref, o_ref):
    agg = q_ref[0] + q_ref[1] + u_ref[...]
    t = jnp.dot(agg * d_ref[...], w_ref[...],
                preferred_element_type=jnp.float32)
    t = t + b_ref[...]
    m = jnp.max(t, axis=1, keepdims=True)
    e = t - m
    o_ref[...] = e - jnp.log(jnp.sum(jnp.exp(e), axis=1, keepdims=True))


_tc3 = pl.pallas_call(
    _tc3_body,
    grid=(GRID,),
    in_specs=[
        pl.BlockSpec((NC, BN, L), lambda i: (0, i, 0)),
        pl.BlockSpec((BN, L), lambda i: (i, 0)),
        pl.BlockSpec((BN, L), lambda i: (i, 0)),
        pl.BlockSpec((L, 64), lambda i: (0, 0)),
        pl.BlockSpec((1, 64), lambda i: (0, 0)),
    ],
    out_specs=pl.BlockSpec((BN, 64), lambda i: (i, 0)),
    out_shape=jax.ShapeDtypeStruct((N, 64), jnp.float32),
)


def kernel(x, edge_index, W1, b1, W2, b2):
    spread = N + (jnp.arange(EP - E, dtype=jnp.int32) % (NP - N))
    row3 = jnp.concatenate([edge_index[0], spread]).reshape(NW, EPT)
    col3 = jnp.concatenate([edge_index[1], spread]).reshape(NW, NCH, CHUNK)
    ones_src = jnp.ones((CHUNK,), jnp.float32)

    dpart = _sc_deg(ones_src, col3)
    h1 = _tc_mm(x, W1)
    h1p = jnp.pad(h1, ((0, NP - N), (0, 0)))
    p, h1s, dv = _sc_agg1(h1p, dpart, row3, col3)
    q, u = _sc_agg2(p, h1s, dv, b1, row3, col3)
    return _tc3(q, u, dv, W2, b2.reshape(1, 64))
